# Initial kernel scaffold; baseline (speedup 1.0000x reference)
import jax
import jax.numpy as jnp
from jax.experimental import pallas as pl


def kernel(x, Wq, bq, Wk, bk, Wv, bv, Wo, bo):
    raise NotImplementedError("write your pallas kernel here")


# speedup vs baseline: 2.3702x; 2.3702x over previous
import functools
import math

import jax
import jax.numpy as jnp
from jax import lax
from jax.experimental import pallas as pl
from jax.experimental.pallas import tpu as pltpu

H = 16


def _qkv_body(x_ref, wqt_ref, wkt_ref, wvt_ref, bq_ref, bk_ref, bv_ref,
              q_ref, k_ref, v_ref, qn_ref):
    xb = x_ref[:]
    d = xb.shape[1]
    hd = d // H
    q = jnp.dot(xb, wqt_ref[:], preferred_element_type=jnp.float32) + bq_ref[:]
    q_ref[:] = q
    k_ref[:] = jnp.dot(xb, wkt_ref[:], preferred_element_type=jnp.float32) + bk_ref[:]
    v_ref[:] = jnp.dot(xb, wvt_ref[:], preferred_element_type=jnp.float32) + bv_ref[:]
    seg = (lax.broadcasted_iota(jnp.int32, (d, H), 0) // hd
           == lax.broadcasted_iota(jnp.int32, (d, H), 1)).astype(jnp.float32)
    qn_ref[:] = jnp.dot(jnp.abs(q), seg, preferred_element_type=jnp.float32,
                        precision=lax.Precision.HIGHEST)


def _topk_body(qn_ref, idx_ref, qnw_ref, *, u, u_pad):
    t = qn_ref.shape[0]
    qnw_ref[:] = qn_ref[:]
    col = lax.broadcasted_iota(jnp.int32, (t, H), 0)
    lane = lax.broadcasted_iota(jnp.int32, (H, u_pad), 1)

    def step(i, acc):
        qn = qnw_ref[:]
        m = jnp.max(qn, axis=0)
        eq = qn == m[None, :]
        am = jnp.min(jnp.where(eq, col, t), axis=0)
        qnw_ref[:] = jnp.where(col == am[None, :], -jnp.inf, qn)
        return acc + jnp.where(lane == i, am[:, None], 0)

    idx_ref[:] = lax.fori_loop(0, u, step, jnp.zeros((H, u_pad), jnp.int32))


def _attn_body(q_ref, k_ref, v_ref, idx_ref, o_ref, *, u_pad, scale, hd):
    g = pl.program_id(0)
    t = k_ref.shape[0]
    for p in range(2):
        idx_row = idx_ref[pl.ds(2 * g + p, 1), :]
        oh = (lax.broadcasted_iota(jnp.int32, (t, u_pad), 0) == idx_row
              ).astype(jnp.float32)
        qh = q_ref[:, p * hd:(p + 1) * hd]
        kh = k_ref[:, p * hd:(p + 1) * hd]
        vh = v_ref[:, p * hd:(p + 1) * hd]
        qsel = lax.dot_general(oh, qh, (((0,), (0,)), ((), ())),
                               preferred_element_type=jnp.float32,
                               precision=lax.Precision.HIGHEST)
        scores = lax.dot_general(qsel, kh, (((1,), (1,)), ((), ())),
                                 preferred_element_type=jnp.float32) * scale
        tpos = lax.broadcasted_iota(jnp.int32, (t, 1), 0).astype(jnp.float32)
        sel_pos = lax.dot_general(oh, tpos, (((0,), (0,)), ((), ())),
                                  preferred_element_type=jnp.float32, precision=lax.Precision.HIGHEST)
        j = lax.broadcasted_iota(jnp.int32, (u_pad, t), 1).astype(jnp.float32)
        scores = jnp.where(j <= sel_pos, scores, -jnp.inf)
        m = jnp.max(scores, axis=1, keepdims=True)
        e = jnp.exp(scores - m)
        s = jnp.sum(e, axis=1, keepdims=True)
        o_ref[p] = jnp.dot(e / s, vh, preferred_element_type=jnp.float32)


def _out_body(idxf_ref, sel_ref, wot_ref, bo_ref, o_ref, c_ref, *, u, u_pad):
    s = pl.program_id(0)
    bt, d = o_ref.shape
    hd = d // H
    n = H * u_pad

    @pl.when(s == 0)
    def _proj():
        for h in range(H):
            c_ref[pl.ds(h * u_pad, u_pad), :] = jnp.dot(
                sel_ref[pl.ds(h * u_pad, u_pad), :],
                wot_ref[pl.ds(h * hd, hd), :],
                preferred_element_type=jnp.float32,
                precision=lax.Precision.HIGHEST)

    rows = s * bt + lax.broadcasted_iota(jnp.int32, (bt, n), 0)
    pad = lax.broadcasted_iota(jnp.int32, (bt, n), 1) % u_pad < u
    oh = ((rows == idxf_ref[:]) & pad).astype(jnp.float32)
    o_ref[:] = jnp.dot(oh, c_ref[:],
                       preferred_element_type=jnp.float32) + bo_ref[:]


def kernel(x, Wq, bq, Wk, bk, Wv, bv, Wo, bo):
    b, t, d = x.shape
    hd = d // H
    u = min(int(5 * math.log(t)), t)
    u_pad = ((u + 7) // 8) * 8
    scale = hd ** -0.5
    x2 = x.reshape(t, d)
    wqt, wkt, wvt, wot = Wq.T, Wk.T, Wv.T, Wo.T
    bq2, bk2, bv2, bo2 = (z.reshape(1, d) for z in (bq, bk, bv, bo))

    bt = 256
    q, k, v, qn = pl.pallas_call(
        _qkv_body,
        grid=(t // bt,),
        in_specs=[
            pl.BlockSpec((bt, d), lambda i: (i, 0)),
            pl.BlockSpec((d, d), lambda i: (0, 0)),
            pl.BlockSpec((d, d), lambda i: (0, 0)),
            pl.BlockSpec((d, d), lambda i: (0, 0)),
            pl.BlockSpec((1, d), lambda i: (0, 0)),
            pl.BlockSpec((1, d), lambda i: (0, 0)),
            pl.BlockSpec((1, d), lambda i: (0, 0)),
        ],
        out_specs=[
            pl.BlockSpec((bt, d), lambda i: (i, 0)),
            pl.BlockSpec((bt, d), lambda i: (i, 0)),
            pl.BlockSpec((bt, d), lambda i: (i, 0)),
            pl.BlockSpec((bt, H), lambda i: (i, 0)),
        ],
        out_shape=[
            jax.ShapeDtypeStruct((t, d), jnp.float32),
            jax.ShapeDtypeStruct((t, d), jnp.float32),
            jax.ShapeDtypeStruct((t, d), jnp.float32),
            jax.ShapeDtypeStruct((t, H), jnp.float32),
        ],
    )(x2, wqt, wkt, wvt, bq2, bk2, bv2)

    idx = pl.pallas_call(
        functools.partial(_topk_body, u=u, u_pad=u_pad),
        grid=(1,),
        in_specs=[pl.BlockSpec((t, H), lambda i: (0, 0))],
        out_specs=pl.BlockSpec((H, u_pad), lambda i: (0, 0)),
        out_shape=jax.ShapeDtypeStruct((H, u_pad), jnp.int32),
        scratch_shapes=[pltpu.VMEM((t, H), jnp.float32)],
    )(qn)

    sel = pl.pallas_call(
        functools.partial(_attn_body, u_pad=u_pad, scale=scale, hd=hd),
        grid=(H // 2,),
        in_specs=[
            pl.BlockSpec((t, 2 * hd), lambda g: (0, g)),
            pl.BlockSpec((t, 2 * hd), lambda g: (0, g)),
            pl.BlockSpec((t, 2 * hd), lambda g: (0, g)),
            pl.BlockSpec((H, u_pad), lambda g: (0, 0)),
        ],
        out_specs=pl.BlockSpec((2, u_pad, hd), lambda g: (g, 0, 0)),
        out_shape=jax.ShapeDtypeStruct((H, u_pad, hd), jnp.float32),
    )(q, k, v, idx)

    out = pl.pallas_call(
        functools.partial(_out_body, u=u, u_pad=u_pad),
        grid=(t // bt,),
        in_specs=[
            pl.BlockSpec((1, H * u_pad), lambda i: (0, 0)),
            pl.BlockSpec((H * u_pad, hd), lambda i: (0, 0)),
            pl.BlockSpec((d, d), lambda i: (0, 0)),
            pl.BlockSpec((1, d), lambda i: (0, 0)),
        ],
        out_specs=pl.BlockSpec((bt, d), lambda i: (i, 0)),
        out_shape=jax.ShapeDtypeStruct((t, d), jnp.float32),
        scratch_shapes=[pltpu.VMEM((H * u_pad, d), jnp.float32)],
    )(idx.reshape(1, H * u_pad), sel.reshape(H * u_pad, hd), wot, bo2)
    return out.reshape(b, t, d)


# speedup vs baseline: 2.8476x; 1.2014x over previous
import functools
import math

import jax
import jax.numpy as jnp
from jax import lax
from jax.experimental import pallas as pl
from jax.experimental.pallas import tpu as pltpu

H = 16


def _qn_topk_body(x_ref, wq_ref, bq_ref, idx_ref, qn_ref, qnw_ref,
                  *, bt, u, u_pad):
    s = pl.program_id(0)
    nb = pl.num_programs(0) - 1
    t, d = x_ref.shape
    hd = d // H

    @pl.when(s < nb)
    def _qn():
        xb = x_ref[pl.ds(s * bt, bt), :]
        q = lax.dot_general(xb, wq_ref[:], (((1,), (1,)), ((), ())),
                            preferred_element_type=jnp.float32) + bq_ref[:]
        v = jnp.abs(q).reshape(bt, H, hd)
        w = hd
        while w > 1:
            v = v[:, :, :w // 2] + v[:, :, w // 2:w]
            w //= 2
        qn_ref[pl.ds(s * bt, bt), :] = v.reshape(bt, H)

    @pl.when(s == nb)
    def _topk():
        pt = t // nb
        qnw_ref[:] = jnp.concatenate(
            [qn_ref[pl.ds(p * pt, pt), :] for p in range(nb)], axis=1)
        ncol = nb * H
        col2 = lax.broadcasted_iota(jnp.int32, (pt, ncol), 0)
        rbase = (lax.broadcasted_iota(jnp.int32, (u_pad, ncol), 1) // H) * pt
        row40 = lax.broadcasted_iota(jnp.int32, (u_pad, ncol), 0)

        def step1(i, carry):
            cv, ci = carry
            qn = qnw_ref[:]
            m = jnp.max(qn, axis=0)
            eq = qn == m[None, :]
            am = jnp.min(jnp.where(eq, col2, pt), axis=0)
            qnw_ref[:] = jnp.where(col2 == am[None, :], -jnp.inf, qn)
            hit = row40 == i
            cv = jnp.where(hit, m[None, :], cv)
            ci = jnp.where(hit, am[None, :], ci)
            return cv, ci

        cv, ci = lax.fori_loop(
            0, u, step1,
            (jnp.full((u_pad, ncol), -jnp.inf, jnp.float32),
             jnp.zeros((u_pad, ncol), jnp.int32)))
        ci = ci + rbase
        cvals = jnp.concatenate(
            [cv[:, p * H:(p + 1) * H] for p in range(nb)], axis=0)
        cidx = jnp.concatenate(
            [ci[:, p * H:(p + 1) * H] for p in range(nb)], axis=0)
        nc = nb * u_pad
        col3 = lax.broadcasted_iota(jnp.int32, (nc, H), 0)
        lane = lax.broadcasted_iota(jnp.int32, (H, u_pad), 1)

        def step2(i, carry):
            acc, vals = carry
            m = jnp.max(vals, axis=0)
            eq = vals == m[None, :]
            am = jnp.min(jnp.where(eq, col3, nc), axis=0)
            pick = col3 == am[None, :]
            g = jnp.sum(jnp.where(pick, cidx, 0), axis=0)
            vals = jnp.where(pick, -jnp.inf, vals)
            acc = acc + jnp.where(lane == i, g[:, None], 0)
            return acc, vals

        acc, _ = lax.fori_loop(
            0, u, step2, (jnp.zeros((H, u_pad), jnp.int32), cvals))
        idx_ref[:] = acc


def _attn_body(x_ref, wq_ref, wk_ref, wv_ref, bqr_ref, bkr_ref, bvr_ref,
               idxs_ref, idxv_ref, o_ref, xs_ref, qk_ref, sc_ref, av_ref,
               *, u, u_pad, scale, hd):
    t, d = x_ref.shape
    eye = (lax.broadcasted_iota(jnp.int32, (u_pad, u_pad), 0)
           == lax.broadcasted_iota(jnp.int32, (u_pad, u_pad), 1)
           ).astype(jnp.float32)
    j_iota = lax.broadcasted_iota(jnp.int32, (u_pad, t), 1).astype(jnp.float32)
    for h in range(H):
        for i in range(u_pad):
            r = idxs_ref[h, i]
            xs_ref[h * u_pad + i:h * u_pad + i + 1, :] = x_ref[pl.ds(r, 1), :]
    for h in range(H):
        xs = xs_ref[pl.ds(h * u_pad, u_pad), :]
        qsel = lax.dot_general(
            xs, wq_ref[pl.ds(h * hd, hd), :], (((1,), (1,)), ((), ())),
            preferred_element_type=jnp.float32) + bqr_ref[pl.ds(h, 1), :]
        qk_ref[pl.ds(h * u_pad, u_pad), :] = lax.dot_general(
            qsel, wk_ref[pl.ds(h * hd, hd), :], (((1,), (0,)), ((), ())),
            preferred_element_type=jnp.float32)
        sbias = lax.dot_general(qsel, bkr_ref[pl.ds(h, 1), :],
                                (((1,), (1,)), ((), ())),
                                preferred_element_type=jnp.float32)
        sc_ref[pl.ds(h * u_pad, u_pad), 0:1] = sbias
    sb_all = sc_ref[:, 0:1]
    scores_all = (lax.dot_general(qk_ref[:], x_ref[:], (((1,), (1,)), ((), ())),
                                  preferred_element_type=jnp.float32)
                  + sb_all) * scale
    sc_ref[:] = scores_all
    for h in range(H):
        scores = sc_ref[pl.ds(h * u_pad, u_pad), :]
        idx_row = idxv_ref[pl.ds(h, 1), :].astype(jnp.float32)
        sel_pos = lax.dot_general(eye, idx_row, (((1,), (1,)), ((), ())),
                                  preferred_element_type=jnp.float32,
                                  precision=lax.Precision.HIGHEST)
        scores = jnp.where(j_iota <= sel_pos, scores, -jnp.inf)
        m = jnp.max(scores, axis=1, keepdims=True)
        e = jnp.exp(scores - m)
        sc_ref[pl.ds(h * u_pad, u_pad), :] = e / jnp.sum(e, axis=1,
                                                         keepdims=True)
    av_ref[:] = lax.dot_general(sc_ref[:], x_ref[:], (((1,), (0,)), ((), ())),
                                preferred_element_type=jnp.float32)
    for h in range(H):
        o_ref[h] = lax.dot_general(
            av_ref[pl.ds(h * u_pad, u_pad), :], wv_ref[pl.ds(h * hd, hd), :],
            (((1,), (1,)), ((), ())),
            preferred_element_type=jnp.float32) + bvr_ref[pl.ds(h, 1), :]


def _out_body(idxf_ref, sel_ref, wot_ref, bo_ref, o_ref, c_ref, *, u, u_pad):
    s = pl.program_id(0)
    bt, d = o_ref.shape
    hd = d // H
    n = H * u_pad

    @pl.when(s == 0)
    def _proj():
        for h in range(H):
            c_ref[pl.ds(h * u_pad, u_pad), :] = jnp.dot(
                sel_ref[pl.ds(h * u_pad, u_pad), :],
                wot_ref[pl.ds(h * hd, hd), :],
                preferred_element_type=jnp.float32,
                precision=lax.Precision.HIGHEST)

    rows = s * bt + lax.broadcasted_iota(jnp.int32, (bt, n), 0)
    pad = lax.broadcasted_iota(jnp.int32, (bt, n), 1) % u_pad < u
    oh = ((rows == idxf_ref[:]) & pad).astype(jnp.float32)
    o_ref[:] = jnp.dot(oh, c_ref[:],
                       preferred_element_type=jnp.float32) + bo_ref[:]


def kernel(x, Wq, bq, Wk, bk, Wv, bv, Wo, bo):
    b, t, d = x.shape
    hd = d // H
    u = min(int(5 * math.log(t)), t)
    u_pad = ((u + 7) // 8) * 8
    scale = hd ** -0.5
    x2 = x.reshape(t, d)
    wot = Wo.T
    bq2, bo2 = bq.reshape(1, d), bo.reshape(1, d)
    bqr, bkr, bvr = (z.reshape(H, hd) for z in (bq, bk, bv))

    bt = 256
    nb = t // bt
    idx = pl.pallas_call(
        functools.partial(_qn_topk_body, bt=bt, u=u, u_pad=u_pad),
        grid=(nb + 1,),
        in_specs=[
            pl.BlockSpec((t, d), lambda i: (0, 0)),
            pl.BlockSpec((d, d), lambda i: (0, 0)),
            pl.BlockSpec((1, d), lambda i: (0, 0)),
        ],
        out_specs=pl.BlockSpec((H, u_pad), lambda i: (0, 0)),
        out_shape=jax.ShapeDtypeStruct((H, u_pad), jnp.int32),
        scratch_shapes=[pltpu.VMEM((t, H), jnp.float32),
                        pltpu.VMEM((t // nb, nb * H), jnp.float32)],
    )(x2, Wq, bq2)

    sel = pl.pallas_call(
        functools.partial(_attn_body, u=u, u_pad=u_pad, scale=scale, hd=hd),
        grid=(1,),
        in_specs=[
            pl.BlockSpec((t, d), lambda i: (0, 0)),
            pl.BlockSpec((d, d), lambda i: (0, 0)),
            pl.BlockSpec((d, d), lambda i: (0, 0)),
            pl.BlockSpec((d, d), lambda i: (0, 0)),
            pl.BlockSpec((H, hd), lambda i: (0, 0)),
            pl.BlockSpec((H, hd), lambda i: (0, 0)),
            pl.BlockSpec((H, hd), lambda i: (0, 0)),
            pl.BlockSpec(memory_space=pltpu.SMEM),
            pl.BlockSpec((H, u_pad), lambda i: (0, 0)),
        ],
        out_specs=pl.BlockSpec((H, u_pad, hd), lambda i: (0, 0, 0)),
        out_shape=jax.ShapeDtypeStruct((H, u_pad, hd), jnp.float32),
        scratch_shapes=[pltpu.VMEM((H * u_pad, d), jnp.float32),
                        pltpu.VMEM((H * u_pad, d), jnp.float32),
                        pltpu.VMEM((H * u_pad, t), jnp.float32),
                        pltpu.VMEM((H * u_pad, d), jnp.float32)],
    )(x2, Wq, Wk, Wv, bqr, bkr, bvr, idx, idx)

    out = pl.pallas_call(
        functools.partial(_out_body, u=u, u_pad=u_pad),
        grid=(nb,),
        in_specs=[
            pl.BlockSpec((1, H * u_pad), lambda i: (0, 0)),
            pl.BlockSpec((H * u_pad, hd), lambda i: (0, 0)),
            pl.BlockSpec((d, d), lambda i: (0, 0)),
            pl.BlockSpec((1, d), lambda i: (0, 0)),
        ],
        out_specs=pl.BlockSpec((bt, d), lambda i: (i, 0)),
        out_shape=jax.ShapeDtypeStruct((t, d), jnp.float32),
        scratch_shapes=[pltpu.VMEM((H * u_pad, d), jnp.float32)],
    )(idx.reshape(1, H * u_pad), sel.reshape(H * u_pad, hd), wot, bo2)
    return out.reshape(b, t, d)
